# baseline (speedup 1.0000x reference)
import jax
import jax.numpy as jnp
from jax.experimental import pallas as pl
from jax.experimental.pallas import tpu as pltpu


def _mlp_kernel(x_ref, w1t_ref, b1_ref, w2t_ref, b2_ref, w3t_ref, b3_ref, o_ref):
    h1 = jnp.dot(x_ref[...], w1t_ref[...], preferred_element_type=jnp.float32)
    h1 = jnp.maximum(h1 + b1_ref[...], 0.0)
    h2 = jnp.dot(h1, w2t_ref[...], preferred_element_type=jnp.float32)
    h2 = jnp.maximum(h2 + b2_ref[...], 0.0)
    o = jnp.dot(h2, w3t_ref[...], preferred_element_type=jnp.float32)
    o_ref[...] = o + b3_ref[...]


def kernel(x, w1, b1, w2, b2, w3, b3):
    B, F = x.shape
    H1, H2, O = w1.shape[0], w2.shape[0], w3.shape[0]

    TB = min(B, 16384)
    Bp = pl.cdiv(B, TB) * TB
    if Bp != B:
        x = jnp.pad(x, ((0, Bp - B), (0, 0)))

    w3t_wide = jnp.pad(w3.T, ((0, 0), (0, 128 - O)))
    b3_wide = jnp.pad(b3.reshape(1, O), ((0, 0), (0, 128 - O)))

    out = pl.pallas_call(
        _mlp_kernel,
        out_shape=jax.ShapeDtypeStruct((Bp, 128), jnp.float32),
        grid=(Bp // TB,),
        in_specs=[
            pl.BlockSpec((TB, F), lambda i: (i, 0)),
            pl.BlockSpec((F, H1), lambda i: (0, 0)),
            pl.BlockSpec((1, H1), lambda i: (0, 0)),
            pl.BlockSpec((H1, H2), lambda i: (0, 0)),
            pl.BlockSpec((1, H2), lambda i: (0, 0)),
            pl.BlockSpec((H2, 128), lambda i: (0, 0)),
            pl.BlockSpec((1, 128), lambda i: (0, 0)),
        ],
        out_specs=pl.BlockSpec((TB, 128), lambda i: (i, 0)),
        compiler_params=pltpu.CompilerParams(
            dimension_semantics=("parallel",),
            vmem_limit_bytes=96 * 1024 * 1024,
        ),
    )(x, w1.T, b1.reshape(1, H1), w2.T, b2.reshape(1, H2), w3t_wide, b3_wide)

    return out[:B, :O]


# speedup vs baseline: 1.0241x; 1.0241x over previous
import jax
import jax.numpy as jnp
from jax.experimental import pallas as pl
from jax.experimental.pallas import tpu as pltpu


def _mlp_kernel(x_ref, w1t_ref, b1_ref, w2t_ref, b2_ref, w3t_ref, b3_ref, o_ref):
    h1 = jnp.dot(x_ref[...], w1t_ref[...], preferred_element_type=jnp.float32)
    h1 = jnp.maximum(h1 + b1_ref[...], 0.0)
    h2 = jnp.dot(h1, w2t_ref[...], preferred_element_type=jnp.float32)
    h2 = jnp.maximum(h2 + b2_ref[...], 0.0)
    o = jnp.dot(h2, w3t_ref[...], preferred_element_type=jnp.float32)
    o = o + b3_ref[...]
    o_ref[...] = o.reshape(o_ref.shape)


def kernel(x, w1, b1, w2, b2, w3, b3):
    B, F = x.shape
    H1, H2, O = w1.shape[0], w2.shape[0], w3.shape[0]

    TB = min(B, 16384)
    Bp = pl.cdiv(B, TB) * TB
    if Bp != B:
        x = jnp.pad(x, ((0, Bp - B), (0, 0)))

    out = pl.pallas_call(
        _mlp_kernel,
        out_shape=jax.ShapeDtypeStruct((Bp // 8, 8, O), jnp.float32),
        grid=(Bp // TB,),
        in_specs=[
            pl.BlockSpec((TB, F), lambda i: (i, 0)),
            pl.BlockSpec((F, H1), lambda i: (0, 0)),
            pl.BlockSpec((1, H1), lambda i: (0, 0)),
            pl.BlockSpec((H1, H2), lambda i: (0, 0)),
            pl.BlockSpec((1, H2), lambda i: (0, 0)),
            pl.BlockSpec((H2, O), lambda i: (0, 0)),
            pl.BlockSpec((1, O), lambda i: (0, 0)),
        ],
        out_specs=pl.BlockSpec((TB // 8, 8, O), lambda i: (i, 0, 0)),
        compiler_params=pltpu.CompilerParams(
            dimension_semantics=("parallel",),
            vmem_limit_bytes=64 * 1024 * 1024,
        ),
        cost_estimate=pl.CostEstimate(
            flops=2 * B * (F * H1 + H1 * H2 + H2 * O),
            transcendentals=0,
            bytes_accessed=4 * (B * F + B * O + F * H1 + H1 + H1 * H2 + H2 + H2 * O + O),
        ),
    )(x, w1.T, b1.reshape(1, H1), w2.T, b2.reshape(1, H2), w3.T, b3.reshape(1, O))

    return out.reshape(Bp, O)[:B]


# speedup vs baseline: 1.0336x; 1.0092x over previous
import jax
import jax.numpy as jnp
from jax.experimental import pallas as pl
from jax.experimental.pallas import tpu as pltpu


def _dot_t(a, w):
    return jax.lax.dot_general(a, w, (((1,), (1,)), ((), ())),
                               preferred_element_type=jnp.float32)


def _mlp_kernel(x_ref, w1_ref, b1_ref, w2_ref, b2_ref, w3_ref, b3_ref, o_ref):
    h1 = jnp.maximum(_dot_t(x_ref[...], w1_ref[...]) + b1_ref[...].T, 0.0)
    h2 = jnp.maximum(_dot_t(h1, w2_ref[...]) + b2_ref[...].T, 0.0)
    o = _dot_t(h2, w3_ref[...]) + b3_ref[...].T
    o_ref[...] = o.reshape(o_ref.shape)


def kernel(x, w1, b1, w2, b2, w3, b3):
    B, F = x.shape
    H1, H2, O = w1.shape[0], w2.shape[0], w3.shape[0]

    TB = min(B, 16384)
    Bp = pl.cdiv(B, TB) * TB
    if Bp != B:
        x = jnp.pad(x, ((0, Bp - B), (0, 0)))

    out = pl.pallas_call(
        _mlp_kernel,
        out_shape=jax.ShapeDtypeStruct((Bp // 8, 8, O), jnp.float32),
        grid=(Bp // TB,),
        in_specs=[
            pl.BlockSpec((TB, F), lambda i: (i, 0)),
            pl.BlockSpec((H1, F), lambda i: (0, 0)),
            pl.BlockSpec((H1, 1), lambda i: (0, 0)),
            pl.BlockSpec((H2, H1), lambda i: (0, 0)),
            pl.BlockSpec((H2, 1), lambda i: (0, 0)),
            pl.BlockSpec((O, H2), lambda i: (0, 0)),
            pl.BlockSpec((O, 1), lambda i: (0, 0)),
        ],
        out_specs=pl.BlockSpec((TB // 8, 8, O), lambda i: (i, 0, 0)),
        compiler_params=pltpu.CompilerParams(
            dimension_semantics=("parallel",),
            vmem_limit_bytes=64 * 1024 * 1024,
        ),
        cost_estimate=pl.CostEstimate(
            flops=2 * B * (F * H1 + H1 * H2 + H2 * O),
            transcendentals=0,
            bytes_accessed=4 * (B * F + B * O + F * H1 + H1 + H1 * H2 + H2 + H2 * O + O),
        ),
    )(x, w1, b1, w2, b2, w3, b3)

    return out.reshape(Bp, O)[:B]
